# baseline (speedup 1.0000x reference)
import jax
import jax.numpy as jnp
from jax.experimental import pallas as pl
from jax.experimental.pallas import tpu as pltpu

TOKENS = 32768
D_MODEL = 768
N_EXPERTS = 8
TOP_K = 2

CT = 1024
NBUF = 4
NCHUNK = TOKENS // CT


def _router_body(x_hbm, w_ref, s_ref, ew_ref, ei_ref, xbuf, sems):
    i = pl.program_id(0)

    def issue(chunk, slot):
        pltpu.make_async_copy(
            x_hbm.at[pl.ds(chunk * CT, CT), :], xbuf.at[slot], sems.at[slot]
        ).start()

    @pl.when(i == 0)
    def _prologue():
        for b in range(NBUF - 1):
            issue(b, b)

    @pl.when(i + (NBUF - 1) < NCHUNK)
    def _prefetch():
        issue(i + (NBUF - 1), jax.lax.rem(i + (NBUF - 1), NBUF))

    slot = jax.lax.rem(i, NBUF)
    pltpu.make_async_copy(
        x_hbm.at[pl.ds(i * CT, CT), :], xbuf.at[slot], sems.at[slot]
    ).wait()

    x = xbuf[slot]
    w = w_ref[...]
    lt = jax.lax.dot_general(
        w, x, (((1,), (1,)), ((), ())), preferred_element_type=jnp.float32
    )
    m = jnp.max(lt, axis=0, keepdims=True)
    e = jnp.exp(lt - m)
    p = e / jnp.sum(e, axis=0, keepdims=True)
    s_ref[...] = p.T

    neg = jnp.float32(-1.0)
    m1 = jnp.full((1, CT), neg, jnp.float32)
    m2 = jnp.full((1, CT), neg, jnp.float32)
    i1 = jnp.zeros((1, CT), jnp.int32)
    i2 = jnp.zeros((1, CT), jnp.int32)
    for ei in range(N_EXPERTS):
        v = p[ei : ei + 1, :]
        ec = jnp.full((1, CT), ei, jnp.int32)
        gt1 = v > m1
        gt2 = v > m2
        i2 = jnp.where(gt1, i1, jnp.where(gt2, ec, i2))
        m2 = jnp.where(gt1, m1, jnp.where(gt2, v, m2))
        i1 = jnp.where(gt1, ec, i1)
        m1 = jnp.where(gt1, v, m1)
    ew_ref[...] = jnp.concatenate([m1, m2], axis=0).T
    ei_ref[...] = jnp.concatenate([i1, i2], axis=0).T


def kernel(x, W):
    grid = (NCHUNK,)
    scores, ew, ei = pl.pallas_call(
        _router_body,
        grid=grid,
        in_specs=[
            pl.BlockSpec(memory_space=pl.ANY),
            pl.BlockSpec((N_EXPERTS, D_MODEL), lambda i: (0, 0)),
        ],
        out_specs=[
            pl.BlockSpec((CT, N_EXPERTS), lambda i: (i, 0)),
            pl.BlockSpec((CT, TOP_K), lambda i: (i, 0)),
            pl.BlockSpec((CT, TOP_K), lambda i: (i, 0)),
        ],
        out_shape=[
            jax.ShapeDtypeStruct((TOKENS, N_EXPERTS), jnp.float32),
            jax.ShapeDtypeStruct((TOKENS, TOP_K), jnp.float32),
            jax.ShapeDtypeStruct((TOKENS, TOP_K), jnp.int32),
        ],
        scratch_shapes=[
            pltpu.VMEM((NBUF, CT, D_MODEL), jnp.float32),
            pltpu.SemaphoreType.DMA((NBUF,)),
        ],
        compiler_params=pltpu.CompilerParams(
            dimension_semantics=("arbitrary",),
        ),
    )(x, W)
    return (scores, ew, ei)
